# baseline (speedup 1.0000x reference)
import functools

import jax
import jax.numpy as jnp
from jax.experimental import pallas as pl
from jax.experimental.pallas import tpu as pltpu

N = 10000
D_IN = 128
D_OUT = 128
BLK = 80


def _decoder_kernel(x_ref, w_ref, b_ref, a_ref, out_ref, support, gcn, m,
                    *, nsteps):
    i = pl.program_id(0)

    @pl.when(i == 0)
    def _init():
        support[...] = jnp.dot(x_ref[...], w_ref[...],
                               preferred_element_type=jnp.float32)
        m[...] = jnp.full_like(m[...], -jnp.inf)

    g = jnp.dot(a_ref[...], support[...],
                preferred_element_type=jnp.float32) + b_ref[...]
    gcn[pl.ds(i * BLK, BLK), :] = g
    m[...] = jnp.maximum(m[...], jnp.max(g, axis=0, keepdims=True))

    @pl.when(i == nsteps - 1)
    def _flush():
        e = jnp.exp(gcn[...] - m[...])
        z = jnp.sum(e, axis=0, keepdims=True)
        acc = jax.lax.dot_general(e, x_ref[...], (((0,), (0,)), ((), ())),
                                  preferred_element_type=jnp.float32)
        out_ref[...] = acc / z.T


def kernel(main_feat, main_adj, W, b):
    nsteps = N // BLK
    b2d = b.reshape(1, D_OUT)
    out = pl.pallas_call(
        functools.partial(_decoder_kernel, nsteps=nsteps),
        grid=(nsteps,),
        in_specs=[
            pl.BlockSpec((N, D_IN), lambda i: (0, 0)),
            pl.BlockSpec((D_IN, D_OUT), lambda i: (0, 0)),
            pl.BlockSpec((1, D_OUT), lambda i: (0, 0)),
            pl.BlockSpec((BLK, N), lambda i: (i, 0)),
        ],
        out_specs=pl.BlockSpec((D_OUT, D_IN), lambda i: (0, 0)),
        out_shape=jax.ShapeDtypeStruct((D_OUT, D_IN), jnp.float32),
        scratch_shapes=[
            pltpu.VMEM((N, D_OUT), jnp.float32),
            pltpu.VMEM((N, D_OUT), jnp.float32),
            pltpu.VMEM((1, D_OUT), jnp.float32),
        ],
        compiler_params=pltpu.CompilerParams(
            dimension_semantics=("arbitrary",),
        ),
    )(main_feat, W, b2d, main_adj)
    return out


# speedup vs baseline: 1.2953x; 1.2953x over previous
import functools

import jax
import jax.numpy as jnp
from jax.experimental import pallas as pl
from jax.experimental.pallas import tpu as pltpu

N = 10000
D_IN = 128
D_OUT = 128
BLK = 200


def _decoder_kernel(x_ref, w_ref, b_ref, a_ref, xblk_ref, out_ref,
                    support, m, z, acc, *, nsteps):
    i = pl.program_id(0)

    @pl.when(i == 0)
    def _init():
        support[...] = jnp.dot(x_ref[...], w_ref[...],
                               preferred_element_type=jnp.float32)
        m[...] = jnp.full_like(m[...], -jnp.inf)
        z[...] = jnp.zeros_like(z[...])
        acc[...] = jnp.zeros_like(acc[...])

    g = jnp.dot(a_ref[...], support[...],
                preferred_element_type=jnp.float32) + b_ref[...]
    new_m = jnp.maximum(m[...], jnp.max(g, axis=0, keepdims=True))
    alpha = jnp.exp(m[...] - new_m)
    e = jnp.exp(g - new_m)
    upd = jax.lax.dot_general(e, xblk_ref[...], (((0,), (0,)), ((), ())),
                              preferred_element_type=jnp.float32)
    z[...] = z[...] * alpha + jnp.sum(e, axis=0, keepdims=True)
    acc[...] = acc[...] * alpha.T + upd
    m[...] = new_m

    @pl.when(i == nsteps - 1)
    def _flush():
        out_ref[...] = acc[...] / z[...].T


def kernel(main_feat, main_adj, W, b):
    nsteps = N // BLK
    b2d = b.reshape(1, D_OUT)
    out = pl.pallas_call(
        functools.partial(_decoder_kernel, nsteps=nsteps),
        grid=(nsteps,),
        in_specs=[
            pl.BlockSpec((N, D_IN), lambda i: (0, 0)),
            pl.BlockSpec((D_IN, D_OUT), lambda i: (0, 0)),
            pl.BlockSpec((1, D_OUT), lambda i: (0, 0)),
            pl.BlockSpec((BLK, N), lambda i: (i, 0)),
            pl.BlockSpec((BLK, D_IN), lambda i: (i, 0)),
        ],
        out_specs=pl.BlockSpec((D_OUT, D_IN), lambda i: (0, 0)),
        out_shape=jax.ShapeDtypeStruct((D_OUT, D_IN), jnp.float32),
        scratch_shapes=[
            pltpu.VMEM((N, D_OUT), jnp.float32),
            pltpu.VMEM((1, D_OUT), jnp.float32),
            pltpu.VMEM((1, D_OUT), jnp.float32),
            pltpu.VMEM((D_OUT, D_IN), jnp.float32),
        ],
        compiler_params=pltpu.CompilerParams(
            dimension_semantics=("arbitrary",),
        ),
    )(main_feat, W, b2d, main_adj, main_feat)
    return out


# speedup vs baseline: 1.3716x; 1.0589x over previous
import functools

import jax
import jax.numpy as jnp
from jax.experimental import pallas as pl
from jax.experimental.pallas import tpu as pltpu

N = 10000
D_IN = 128
D_OUT = 128
BLK = 200


def _decoder_kernel(x_ref, w_ref, b_ref, a_ref, out_ref, support, gcn, m,
                    *, nsteps):
    i = pl.program_id(0)

    @pl.when(i == 0)
    def _init():
        support[...] = jnp.dot(x_ref[...], w_ref[...],
                               preferred_element_type=jnp.float32)
        m[...] = jnp.full_like(m[...], -jnp.inf)

    g = jnp.dot(a_ref[...], support[...],
                preferred_element_type=jnp.float32) + b_ref[...]
    gcn[pl.ds(i * BLK, BLK), :] = g
    m[...] = jnp.maximum(m[...], jnp.max(g, axis=0, keepdims=True))

    @pl.when(i == nsteps - 1)
    def _flush():
        e = jnp.exp(gcn[...] - m[...])
        z = jnp.sum(e, axis=0, keepdims=True)
        acc = jax.lax.dot_general(e, x_ref[...], (((0,), (0,)), ((), ())),
                                  preferred_element_type=jnp.float32)
        out_ref[...] = acc / z.T


def kernel(main_feat, main_adj, W, b):
    nsteps = N // BLK
    b2d = b.reshape(1, D_OUT)
    out = pl.pallas_call(
        functools.partial(_decoder_kernel, nsteps=nsteps),
        grid=(nsteps,),
        in_specs=[
            pl.BlockSpec((N, D_IN), lambda i: (0, 0)),
            pl.BlockSpec((D_IN, D_OUT), lambda i: (0, 0)),
            pl.BlockSpec((1, D_OUT), lambda i: (0, 0)),
            pl.BlockSpec((BLK, N), lambda i: (i, 0)),
        ],
        out_specs=pl.BlockSpec((D_OUT, D_IN), lambda i: (0, 0)),
        out_shape=jax.ShapeDtypeStruct((D_OUT, D_IN), jnp.float32),
        scratch_shapes=[
            pltpu.VMEM((N, D_OUT), jnp.float32),
            pltpu.VMEM((N, D_OUT), jnp.float32),
            pltpu.VMEM((1, D_OUT), jnp.float32),
        ],
        compiler_params=pltpu.CompilerParams(
            dimension_semantics=("arbitrary",),
        ),
    )(main_feat, W, b2d, main_adj)
    return out
